# baseline (speedup 1.0000x reference)
import functools

import jax
import jax.numpy as jnp
from jax.experimental import pallas as pl
from jax.experimental.pallas import tpu as pltpu
from jax.experimental.pallas import tpu_sc as plsc

_LEN = 32
_HID = 8192
_K = 64
_OUT = 64
_NCOL = _HID // _K

_SC_WIN = 256


def _ce(P, i, p):
    a, b = P[i], P[p]
    P[i] = jnp.maximum(a, b)
    P[p] = jnp.minimum(a, b)


def _bitonic_sort_positions(P):
    n = len(P)
    k = 2
    while k <= n:
        j = k // 2
        while j >= 1:
            for i in range(n):
                p = i ^ j
                if p > i:
                    if (i & k) == 0 or k == n:
                        _ce(P, i, p)
                    else:
                        _ce(P, p, i)
            j //= 2
        k *= 2
    return P


def _merge_positions(P):
    n = len(P)
    j = n // 2
    while j >= 1:
        for i in range(n):
            p = i ^ j
            if p > i:
                _ce(P, i, p)
        j //= 2
    return P


def _sub_mask(w, width, lanes):
    return jax.lax.broadcasted_iota(jnp.int32, (width, lanes), 0) < w


def _body_a(x_ref, w_ref, da_ref, *, rb):
    hT = jax.lax.dot_general(
        w_ref[...], x_ref[...],
        dimension_numbers=(((0,), (1,)), ((), ())),
        preferred_element_type=jnp.float32,
    )
    P = [hT[_NCOL * i:_NCOL * (i + 1), :] for i in range(_K)]
    lm = _sub_mask(_NCOL // 2, _NCOL, rb)
    P = [jnp.where(lm, v, -v) for v in P]
    P = _bitonic_sort_positions(P)
    w = _NCOL
    while w > 2:
        half = w // 2
        P = [jnp.maximum(v[:half, :], -v[half:w, :]) for v in P]
        if half > 1:
            lm = _sub_mask(half // 2, half, rb)
            P = [jnp.where(lm, v, -v) for v in P]
        P = _merge_positions(P)
        w = half
    da_ref[0] = jnp.concatenate([v[0:1, :] for v in P], axis=0)
    da_ref[1] = jnp.concatenate([-v[1:2, :] for v in P], axis=0)


def _sc_body(desc_ref, asc_ref, out_ref, in_d, in_a, out_buf):
    c = jax.lax.axis_index("c")
    s = jax.lax.axis_index("s")
    wid = c * 16 + s
    ntok = desc_ref.shape[0]
    ncols = desc_ref.shape[2]
    blocks_per_tok = 32 // ntok
    cols_per = ncols // blocks_per_tok
    tok = wid // blocks_per_tok
    base = (wid % blocks_per_tok) * cols_per
    nwin = cols_per // _SC_WIN

    def win_step(w, _):
        start = base + w * _SC_WIN
        pltpu.sync_copy(desc_ref.at[tok, :, pl.ds(start, _SC_WIN)], in_d)
        pltpu.sync_copy(asc_ref.at[tok, :, pl.ds(start, _SC_WIN)], in_a)

        def grp_step(g, _):
            off = g * 16
            P = [
                jnp.maximum(in_d[i, pl.ds(off, 16)], in_a[i, pl.ds(off, 16)])
                for i in range(_K)
            ]
            P = _merge_positions(P)
            for i in range(_K):
                out_buf[i, pl.ds(off, 16)] = P[i]
            return 0

        jax.lax.fori_loop(0, _SC_WIN // 16, grp_step, 0)
        pltpu.sync_copy(out_buf, out_ref.at[tok, :, pl.ds(start, _SC_WIN)])
        return 0

    jax.lax.fori_loop(0, nwin, win_step, 0)


def _sc_merge(desc, asc):
    sc = pl.kernel(
        _sc_body,
        out_type=jax.ShapeDtypeStruct(desc.shape, jnp.float32),
        mesh=plsc.VectorSubcoreMesh(
            core_axis_name="c", subcore_axis_name="s", num_cores=2, num_subcores=16
        ),
        scratch_types=[
            pltpu.MemorySpace.VMEM((_K, _SC_WIN), jnp.float32),
            pltpu.MemorySpace.VMEM((_K, _SC_WIN), jnp.float32),
            pltpu.MemorySpace.VMEM((_K, _SC_WIN), jnp.float32),
        ],
    )
    return sc(desc, asc)


def _body_b(sg_ref, dk_ref, db_ref, g_ref, b_ref, o_ref):
    dT = jax.lax.dot_general(
        dk_ref[...], sg_ref[...],
        dimension_numbers=(((0,), (0,)), ((), ())),
        preferred_element_type=jnp.float32,
    )
    dT = dT + db_ref[...]
    dT = 0.5 * dT * (1.0 + jax.lax.erf(dT * (2.0 ** -0.5)))
    mu = jnp.mean(dT, axis=0, keepdims=True)
    c = dT - mu
    var = jnp.mean(c * c, axis=0, keepdims=True)
    o_ref[...] = c * jax.lax.rsqrt(var + 1e-6) * g_ref[...] + b_ref[...]


def kernel(inputs, W, Dk, Db, gamma, beta):
    B, LEN = inputs.shape
    T = W.shape[0]
    rb = 128 if B % 128 == 0 else B
    grid = (T, B // rb)
    da = pl.pallas_call(
        functools.partial(_body_a, rb=rb),
        grid=grid,
        in_specs=[
            pl.BlockSpec((rb, LEN), lambda t, i: (i, 0)),
            pl.BlockSpec((None, LEN, _HID), lambda t, i: (t, 0, 0)),
        ],
        out_specs=pl.BlockSpec((2, None, _K, rb), lambda t, i: (0, t, 0, i)),
        out_shape=jax.ShapeDtypeStruct((2, T, _K, B), jnp.float32),
    )(inputs, W)
    sg = _sc_merge(da[0], da[1])
    rb2 = 512 if B % 512 == 0 else B
    out = pl.pallas_call(
        _body_b,
        grid=(T, B // rb2),
        in_specs=[
            pl.BlockSpec((None, _K, rb2), lambda t, i: (t, 0, i)),
            pl.BlockSpec((None, _K, _OUT), lambda t, i: (t, 0, 0)),
            pl.BlockSpec((None, _OUT, 1), lambda t, i: (t, 0, 0)),
            pl.BlockSpec((None, _OUT, 1), lambda t, i: (t, 0, 0)),
            pl.BlockSpec((None, _OUT, 1), lambda t, i: (t, 0, 0)),
        ],
        out_specs=pl.BlockSpec((None, _OUT, rb2), lambda t, i: (t, 0, i)),
        out_shape=jax.ShapeDtypeStruct((T, _OUT, B), jnp.float32),
    )(sg, Dk, Db[:, :, None], gamma[:, :, None], beta[:, :, None])
    return jnp.transpose(out, (2, 0, 1))
